# baseline (speedup 1.0000x reference)
import math

import jax
import jax.numpy as jnp
from jax.experimental import pallas as pl
from jax.experimental.pallas import tpu as pltpu

_J = 16
_R = _J * 128
_N_REAL = 800
_NCH = _N_REAL // _J
_N_ALL = 1000
_HW = 128 * 128
_LN2 = math.log(2.0)
_LOG2E = 1.0 / _LN2


def _bce_body(wts_ref, p_hbm, m_hbm, s_ref, o_ref, acc_ref, cnt_ref):
    acc_ref[...] = jnp.zeros((128, 128), jnp.float32)
    cnt_ref[0] = 0

    def _chunk(p_blk, m_blk):
        c = cnt_ref[0]

        def _inner(j, a):
            x = p_blk[pl.ds(j * 128, 128), :]
            m = m_blk[pl.ds(j * 128, 128), :]
            w = wts_ref[c * _J + j] > 0
            xs = x * _LOG2E
            t = jnp.log2(1.0 + jnp.exp2(xs))
            u = t - jnp.where(m >= 0.5, xs, 0.0)
            return a + jnp.where(w, u, 0.0)

        acc_ref[...] = jax.lax.fori_loop(0, _J, _inner, acc_ref[...],
                                         unroll=False)
        cnt_ref[0] = c + 1

    buf = pl.Buffered(buffer_count=6)
    pltpu.emit_pipeline(
        _chunk,
        grid=(_NCH,),
        in_specs=[
            pl.BlockSpec((_R, 128), lambda c: (c, 0), pipeline_mode=buf),
            pl.BlockSpec((_R, 128), lambda c: (c, 0), pipeline_mode=buf),
        ],
    )(p_hbm, m_hbm)

    s = s_ref[...]
    posf = (s > 0.0).astype(jnp.float32)
    flat = (jax.lax.broadcasted_iota(jnp.int32, (8, 128), 0) * 128
            + jax.lax.broadcasted_iota(jnp.int32, (8, 128), 1))
    denom = jnp.sum(posf)
    pad_cnt = jnp.sum(jnp.where(flat >= _N_REAL, posf, 0.0))
    loss_sum = _LN2 * jnp.sum(acc_ref[...])
    loss = (loss_sum + pad_cnt * (_HW * _LN2)) / denom
    o_ref[...] = jnp.reshape(loss, (1, 1))


def kernel(mask_preds, masks, scores):
    preds2 = mask_preds.reshape(_N_REAL * 128, 128)
    masks2 = masks.reshape(_N_ALL * 128, 128)
    scores_f = scores.reshape(-1)

    wts = (scores_f[:_N_REAL] > 0.0).astype(jnp.int32)
    s_pad = jnp.pad(scores_f, (0, 1024 - _N_ALL),
                    constant_values=-1.0).reshape(8, 128)

    grid_spec = pltpu.PrefetchScalarGridSpec(
        num_scalar_prefetch=1,
        grid=(1,),
        in_specs=[
            pl.BlockSpec(memory_space=pl.ANY),
            pl.BlockSpec(memory_space=pl.ANY),
            pl.BlockSpec((8, 128), lambda i, *_: (0, 0)),
        ],
        out_specs=pl.BlockSpec((1, 1), lambda i, *_: (0, 0)),
        scratch_shapes=[
            pltpu.VMEM((128, 128), jnp.float32),
            pltpu.SMEM((1,), jnp.int32),
        ],
    )
    out = pl.pallas_call(
        _bce_body,
        grid_spec=grid_spec,
        out_shape=jax.ShapeDtypeStruct((1, 1), jnp.float32),
    )(wts, preds2, masks2, s_pad)
    return out[0, 0]


# speedup vs baseline: 1.0304x; 1.0304x over previous
import math

import jax
import jax.numpy as jnp
from jax.experimental import pallas as pl
from jax.experimental.pallas import tpu as pltpu

_J = 8
_R = _J * 128
_N_REAL = 800
_NCH = _N_REAL // _J
_N_ALL = 1000
_HW = 128 * 128
_LN2 = math.log(2.0)
_LOG2E = 1.0 / _LN2


def _bce_body(wts_ref, p_hbm, m_hbm, s_ref, o_ref, acc_ref, cnt_ref):
    acc_ref[...] = jnp.zeros((128, 128), jnp.float32)
    cnt_ref[0] = 0

    def _chunk(p_blk, m_blk):
        c = cnt_ref[0]

        def _inner(j, a):
            x = p_blk[pl.ds(j * 128, 128), :]
            m = m_blk[pl.ds(j * 128, 128), :]
            w = wts_ref[c * _J + j] > 0
            xs = x * _LOG2E
            t = jnp.log2(1.0 + jnp.exp2(xs))
            u = t - jnp.where(m >= 0.5, xs, 0.0)
            return a + jnp.where(w, u, 0.0)

        acc_ref[...] = jax.lax.fori_loop(0, _J, _inner, acc_ref[...],
                                         unroll=False)
        cnt_ref[0] = c + 1

    buf = pl.Buffered(buffer_count=8)
    pltpu.emit_pipeline(
        _chunk,
        grid=(_NCH,),
        in_specs=[
            pl.BlockSpec((_R, 128), lambda c: (c, 0), pipeline_mode=buf),
            pl.BlockSpec((_R, 128), lambda c: (c, 0), pipeline_mode=buf),
        ],
    )(p_hbm, m_hbm)

    s = s_ref[...]
    posf = (s > 0.0).astype(jnp.float32)
    flat = (jax.lax.broadcasted_iota(jnp.int32, (8, 128), 0) * 128
            + jax.lax.broadcasted_iota(jnp.int32, (8, 128), 1))
    denom = jnp.sum(posf)
    pad_cnt = jnp.sum(jnp.where(flat >= _N_REAL, posf, 0.0))
    loss_sum = _LN2 * jnp.sum(acc_ref[...])
    loss = (loss_sum + pad_cnt * (_HW * _LN2)) / denom
    o_ref[...] = jnp.reshape(loss, (1, 1))


def kernel(mask_preds, masks, scores):
    preds2 = mask_preds.reshape(_N_REAL * 128, 128)
    masks2 = masks.reshape(_N_ALL * 128, 128)
    scores_f = scores.reshape(-1)

    wts = (scores_f[:_N_REAL] > 0.0).astype(jnp.int32)
    s_pad = jnp.pad(scores_f, (0, 1024 - _N_ALL),
                    constant_values=-1.0).reshape(8, 128)

    grid_spec = pltpu.PrefetchScalarGridSpec(
        num_scalar_prefetch=1,
        grid=(1,),
        in_specs=[
            pl.BlockSpec(memory_space=pl.ANY),
            pl.BlockSpec(memory_space=pl.ANY),
            pl.BlockSpec((8, 128), lambda i, *_: (0, 0)),
        ],
        out_specs=pl.BlockSpec((1, 1), lambda i, *_: (0, 0)),
        scratch_shapes=[
            pltpu.VMEM((128, 128), jnp.float32),
            pltpu.SMEM((1,), jnp.int32),
        ],
    )
    out = pl.pallas_call(
        _bce_body,
        grid_spec=grid_spec,
        out_shape=jax.ShapeDtypeStruct((1, 1), jnp.float32),
    )(wts, preds2, masks2, s_pad)
    return out[0, 0]
